# baseline (speedup 1.0000x reference)
import jax
import jax.numpy as jnp
from jax.experimental import pallas as pl
from jax.experimental.pallas import tpu as pltpu

_TOTAL_TOK = 16384
_OLD_DIM = 256
_HID_DIM = 512
_NEW_DIM = 128
_TILE = 1024


def _mlp_tile(x_ref, w1_ref, b1_ref, w2_ref, b2_ref, o_ref):
    x = x_ref[...].astype(jnp.bfloat16)
    w1 = w1_ref[...].astype(jnp.bfloat16)
    h = jax.lax.dot_general(
        x, w1, (((1,), (0,)), ((), ())), preferred_element_type=jnp.float32
    )
    h = jnp.maximum(h + b1_ref[...], 0.0).astype(jnp.bfloat16)
    w2 = w2_ref[...].astype(jnp.bfloat16)
    o = jax.lax.dot_general(
        h, w2, (((1,), (0,)), ((), ())), preferred_element_type=jnp.float32
    )
    o_ref[...] = o + b2_ref[...]


def kernel(flat, cu_seqlens, W1, b1, W2, b2):
    del cu_seqlens
    b1r = jnp.reshape(b1, (1, _HID_DIM))
    b2r = jnp.reshape(b2, (1, _NEW_DIM))
    grid = (_TOTAL_TOK // _TILE,)
    out = pl.pallas_call(
        _mlp_tile,
        grid=grid,
        in_specs=[
            pl.BlockSpec((_TILE, _OLD_DIM), lambda i: (i, 0)),
            pl.BlockSpec((_OLD_DIM, _HID_DIM), lambda i: (0, 0)),
            pl.BlockSpec((1, _HID_DIM), lambda i: (0, 0)),
            pl.BlockSpec((_HID_DIM, _NEW_DIM), lambda i: (0, 0)),
            pl.BlockSpec((1, _NEW_DIM), lambda i: (0, 0)),
        ],
        out_specs=pl.BlockSpec((_TILE, _NEW_DIM), lambda i: (i, 0)),
        out_shape=jax.ShapeDtypeStruct((_TOTAL_TOK, _NEW_DIM), jnp.float32),
        compiler_params=pltpu.CompilerParams(
            dimension_semantics=("arbitrary",),
        ),
    )(flat, W1, b1r, W2, b2r)
    return out


# speedup vs baseline: 1.2825x; 1.2825x over previous
import jax
import jax.numpy as jnp
from jax.experimental import pallas as pl
from jax.experimental.pallas import tpu as pltpu

_TOTAL_TOK = 16384
_OLD_DIM = 256
_HID_DIM = 512
_NEW_DIM = 128
_TILE = 2048


def _mlp_tile(x_ref, w1_ref, b1_ref, w2_ref, b2_ref, o_ref):
    x = x_ref[...].astype(jnp.bfloat16)
    w1 = w1_ref[...].astype(jnp.bfloat16)
    h = jax.lax.dot_general(
        x, w1, (((1,), (0,)), ((), ())), preferred_element_type=jnp.float32
    )
    h = jnp.maximum(h + b1_ref[...], 0.0).astype(jnp.bfloat16)
    w2 = w2_ref[...].astype(jnp.bfloat16)
    o = jax.lax.dot_general(
        h, w2, (((1,), (0,)), ((), ())), preferred_element_type=jnp.float32
    )
    o_ref[...] = o + b2_ref[...]


def kernel(flat, cu_seqlens, W1, b1, W2, b2):
    del cu_seqlens
    b1r = jnp.reshape(b1, (1, _HID_DIM))
    b2r = jnp.reshape(b2, (1, _NEW_DIM))
    grid = (_TOTAL_TOK // _TILE,)
    out = pl.pallas_call(
        _mlp_tile,
        grid=grid,
        in_specs=[
            pl.BlockSpec((_TILE, _OLD_DIM), lambda i: (i, 0)),
            pl.BlockSpec((_OLD_DIM, _HID_DIM), lambda i: (0, 0)),
            pl.BlockSpec((1, _HID_DIM), lambda i: (0, 0)),
            pl.BlockSpec((_HID_DIM, _NEW_DIM), lambda i: (0, 0)),
            pl.BlockSpec((1, _NEW_DIM), lambda i: (0, 0)),
        ],
        out_specs=pl.BlockSpec((_TILE, _NEW_DIM), lambda i: (i, 0)),
        out_shape=jax.ShapeDtypeStruct((_TOTAL_TOK, _NEW_DIM), jnp.float32),
        compiler_params=pltpu.CompilerParams(
            dimension_semantics=("arbitrary",),
        ),
    )(flat, W1, b1r, W2, b2r)
    return out


# speedup vs baseline: 1.4363x; 1.1199x over previous
import jax
import jax.numpy as jnp
from jax.experimental import pallas as pl
from jax.experimental.pallas import tpu as pltpu

_TOTAL_TOK = 16384
_OLD_DIM = 256
_HID_DIM = 512
_NEW_DIM = 128
_TILE = 4096


def _mlp_tile(x_ref, w1_ref, b1_ref, w2_ref, b2_ref, o_ref):
    x = x_ref[...].astype(jnp.bfloat16)
    w1 = w1_ref[...].astype(jnp.bfloat16)
    h = jax.lax.dot_general(
        x, w1, (((1,), (0,)), ((), ())), preferred_element_type=jnp.float32
    )
    h = jnp.maximum(h + b1_ref[...], 0.0).astype(jnp.bfloat16)
    w2 = w2_ref[...].astype(jnp.bfloat16)
    o = jax.lax.dot_general(
        h, w2, (((1,), (0,)), ((), ())), preferred_element_type=jnp.float32
    )
    o_ref[...] = o + b2_ref[...]


def kernel(flat, cu_seqlens, W1, b1, W2, b2):
    del cu_seqlens
    b1r = jnp.reshape(b1, (1, _HID_DIM))
    b2r = jnp.reshape(b2, (1, _NEW_DIM))
    grid = (_TOTAL_TOK // _TILE,)
    out = pl.pallas_call(
        _mlp_tile,
        grid=grid,
        in_specs=[
            pl.BlockSpec((_TILE, _OLD_DIM), lambda i: (i, 0)),
            pl.BlockSpec((_OLD_DIM, _HID_DIM), lambda i: (0, 0)),
            pl.BlockSpec((1, _HID_DIM), lambda i: (0, 0)),
            pl.BlockSpec((_HID_DIM, _NEW_DIM), lambda i: (0, 0)),
            pl.BlockSpec((1, _NEW_DIM), lambda i: (0, 0)),
        ],
        out_specs=pl.BlockSpec((_TILE, _NEW_DIM), lambda i: (i, 0)),
        out_shape=jax.ShapeDtypeStruct((_TOTAL_TOK, _NEW_DIM), jnp.float32),
        compiler_params=pltpu.CompilerParams(
            dimension_semantics=("arbitrary",),
        ),
    )(flat, W1, b1r, W2, b2r)
    return out
